# baseline (speedup 1.0000x reference)
import functools

import jax
import jax.numpy as jnp
from jax.experimental import pallas as pl
from jax.experimental.pallas import tpu as pltpu
from jax.experimental.pallas import tpu_sc as plsc


_ROW = 128
_N_OUT = 3


def _gather_rows(x_hbm, out_hbm):
    @pl.when(jax.lax.axis_index("c") == 0)
    def _():
        pltpu.sync_copy(x_hbm.at[pl.ds(0, _N_OUT)], out_hbm)


def kernel(x):
    mesh = plsc.ScalarSubcoreMesh(axis_name="c", num_cores=2)
    k = functools.partial(
        pl.kernel,
        mesh=mesh,
        out_type=jax.ShapeDtypeStruct((_N_OUT, _ROW), jnp.float32),
    )(_gather_rows)
    out = k(x)
    return (out[0], out[1], out[2])


# speedup vs baseline: 1.0903x; 1.0903x over previous
import functools

import jax
import jax.numpy as jnp
from jax.experimental import pallas as pl
from jax.experimental.pallas import tpu as pltpu
from jax.experimental.pallas import tpu_sc as plsc


_ROW = 128
_N_OUT = 3


def _gather_rows(x_hbm, out_hbm):
    pltpu.sync_copy(x_hbm.at[pl.ds(0, _N_OUT)], out_hbm)


def kernel(x):
    mesh = plsc.ScalarSubcoreMesh(axis_name="c", num_cores=1)
    k = functools.partial(
        pl.kernel,
        mesh=mesh,
        out_type=jax.ShapeDtypeStruct((_N_OUT, _ROW), jnp.float32),
    )(_gather_rows)
    out = k(x)
    return (out[0], out[1], out[2])
